# baseline (speedup 1.0000x reference)
import functools

import jax
import jax.numpy as jnp
from jax import lax
from jax.experimental import pallas as pl
from jax.experimental.pallas import tpu as pltpu
from jax.experimental.pallas import tpu_sc as plsc

_H = 768
_NS = _H // 16
_EPS = 1e-12
_CHUNK = 32


def _lane_shuffle(v, perm):
    dnums = lax.GatherDimensionNumbers(
        offset_dims=(), collapsed_slice_dims=(0,), start_index_map=(0,))
    return lax.gather(v, perm[:, None], dnums, slice_sizes=(1,),
                      mode=lax.GatherScatterMode.PROMISE_IN_BOUNDS)


def _rsqrt16(x):
    s = jnp.ones((16,), jnp.float32)
    for p in (32, 16, 8, 4, 2, 1):
        w = x * s * s
        up = jnp.float32(2.0 ** (2 * p))
        lo = jnp.float32(2.0 ** (2 - 2 * p))
        s = s * jnp.where(w >= up, jnp.float32(2.0 ** (-p)),
                          jnp.where(w < lo, jnp.float32(2.0 ** p),
                                    jnp.float32(1.0)))
    w = x * s * s
    y = jnp.where(w < 2.0, jnp.float32(0.85), jnp.float32(0.6))
    hw = w * -0.5
    for _ in range(5):
        y = y * (hw * y * y + 1.5)
    return y * s


def _sc_body(x_hbm, ids_hbm, table_hbm, gamma_hbm, beta_hbm, out_hbm,
             ids_v, x_v, seg_v, o_v, gamma_v, beta_v, sem):
    n_tok = x_hbm.shape[0]
    wid = lax.axis_index("s") * 2 + lax.axis_index("c")
    per_w = n_tok // 32
    n_chunks = per_w // _CHUNK

    pltpu.sync_copy(gamma_hbm, gamma_v)
    pltpu.sync_copy(beta_hbm, beta_v)

    def chunk_body(k, _):
        base = wid * per_w + k * _CHUNK
        pltpu.sync_copy(ids_hbm.at[pl.ds(base, _CHUNK)], ids_v)
        pltpu.async_copy(table_hbm.at[ids_v], seg_v, sem).wait()
        pltpu.sync_copy(x_hbm.at[pl.ds(base, _CHUNK)], x_v)

        def tok_body(t, _):
            acc_s = jnp.zeros((16,), jnp.float32)
            acc_q = jnp.zeros((16,), jnp.float32)
            for j in range(_NS):
                sl = pl.ds(j * 16, 16)
                e = x_v[t, sl] + seg_v[t, sl]
                o_v[t, sl] = e
                acc_s = acc_s + e
                acc_q = acc_q + e * e
            for k in (1, 2, 4, 8):
                perm = jnp.arange(16, dtype=jnp.int32) ^ k
                acc_s = acc_s + _lane_shuffle(acc_s, perm)
                acc_q = acc_q + _lane_shuffle(acc_q, perm)
            mean_v = acc_s * (1.0 / _H)
            var_v = acc_q * (1.0 / _H) - mean_v * mean_v
            var_v = jnp.maximum(var_v, 0.0)
            inv = _rsqrt16(var_v + _EPS)
            for j in range(_NS):
                sl = pl.ds(j * 16, 16)
                o_v[t, sl] = (o_v[t, sl] - mean_v) * inv * gamma_v[sl] + beta_v[sl]
            return _

        lax.fori_loop(0, _CHUNK, tok_body, None)
        pltpu.sync_copy(o_v, out_hbm.at[pl.ds(base, _CHUNK)])
        return _

    lax.fori_loop(0, n_chunks, chunk_body, None)


def kernel(input_embs, seg_ids, seg_table, ln_gamma, ln_beta):
    b, s, h = input_embs.shape
    n_tok = b * s
    x = input_embs.reshape(n_tok, h)
    ids = seg_ids.astype(jnp.int32).reshape(n_tok)

    mesh = plsc.VectorSubcoreMesh(core_axis_name="c", subcore_axis_name="s")
    sck = functools.partial(
        pl.kernel,
        mesh=mesh,
        out_type=jax.ShapeDtypeStruct((n_tok, h), jnp.float32),
        scratch_types=[
            pltpu.VMEM((_CHUNK,), jnp.int32),
            pltpu.VMEM((_CHUNK, h), jnp.float32),
            pltpu.VMEM((_CHUNK, h), jnp.float32),
            pltpu.VMEM((_CHUNK, h), jnp.float32),
            pltpu.VMEM((h,), jnp.float32),
            pltpu.VMEM((h,), jnp.float32),
            pltpu.SemaphoreType.DMA,
        ],
    )(_sc_body)
    out = sck(x, ids, seg_table, ln_gamma, ln_beta)
    return out.reshape(b, s, h)


# speedup vs baseline: 3.4950x; 3.4950x over previous
import functools

import jax
import jax.numpy as jnp
from jax import lax
from jax.experimental import pallas as pl
from jax.experimental.pallas import tpu as pltpu
from jax.experimental.pallas import tpu_sc as plsc

_H = 768
_NS = _H // 16
_EPS = 1e-12
_CHUNK = 32
_T = 2048
_SC_TOK = 2048


def _tc_body(ids_ref, x_ref, table_ref, gamma_ref, beta_ref, out_ref):
    ids = ids_ref[0, 0, :]
    x = x_ref[...]
    table = table_ref[...]
    onehot = (ids[:, None] == jax.lax.broadcasted_iota(jnp.int32, (_T, 4), 1))
    seg = jnp.dot(onehot.astype(jnp.float32), table,
                  preferred_element_type=jnp.float32)
    e = x + seg
    mean = jnp.mean(e, axis=1, keepdims=True)
    d = e - mean
    var = jnp.mean(d * d, axis=1, keepdims=True)
    normed = d * jax.lax.rsqrt(var + _EPS)
    out_ref[...] = normed * gamma_ref[...] + beta_ref[...]


def _lane_shuffle(v, perm):
    dnums = lax.GatherDimensionNumbers(
        offset_dims=(), collapsed_slice_dims=(0,), start_index_map=(0,))
    return lax.gather(v, perm[:, None], dnums, slice_sizes=(1,),
                      mode=lax.GatherScatterMode.PROMISE_IN_BOUNDS)


def _rsqrt16(x):
    s = jnp.ones((16,), jnp.float32)
    for p in (32, 16, 8, 4, 2, 1):
        w = x * s * s
        up = jnp.float32(2.0 ** (2 * p))
        lo = jnp.float32(2.0 ** (2 - 2 * p))
        s = s * jnp.where(w >= up, jnp.float32(2.0 ** (-p)),
                          jnp.where(w < lo, jnp.float32(2.0 ** p),
                                    jnp.float32(1.0)))
    w = x * s * s
    y = jnp.where(w < 2.0, jnp.float32(0.85), jnp.float32(0.6))
    hw = w * -0.5
    for _ in range(5):
        y = y * (hw * y * y + 1.5)
    return y * s


def _sc_body(x_hbm, ids_hbm, table_hbm, gamma_hbm, beta_hbm, out_hbm,
             ids_v, x_v, seg_v, o_v, gamma_v, beta_v, sem):
    n_tok = x_hbm.shape[0]
    wid = lax.axis_index("s") * 2 + lax.axis_index("c")
    per_w = n_tok // 32
    n_chunks = per_w // _CHUNK

    pltpu.sync_copy(gamma_hbm, gamma_v)
    pltpu.sync_copy(beta_hbm, beta_v)

    def chunk_body(k, _):
        base = wid * per_w + k * _CHUNK
        pltpu.sync_copy(ids_hbm.at[pl.ds(base, _CHUNK)], ids_v)
        pltpu.async_copy(table_hbm.at[ids_v], seg_v, sem).wait()
        pltpu.sync_copy(x_hbm.at[pl.ds(base, _CHUNK)], x_v)

        def tok_body(t, _):
            acc_s = jnp.zeros((16,), jnp.float32)
            acc_q = jnp.zeros((16,), jnp.float32)
            for j in range(_NS):
                sl = pl.ds(j * 16, 16)
                e = x_v[t, sl] + seg_v[t, sl]
                o_v[t, sl] = e
                acc_s = acc_s + e
                acc_q = acc_q + e * e
            for k2 in (1, 2, 4, 8):
                perm = jnp.arange(16, dtype=jnp.int32) ^ k2
                acc_s = acc_s + _lane_shuffle(acc_s, perm)
                acc_q = acc_q + _lane_shuffle(acc_q, perm)
            mean_v = acc_s * (1.0 / _H)
            var_v = acc_q * (1.0 / _H) - mean_v * mean_v
            var_v = jnp.maximum(var_v, 0.0)
            inv = _rsqrt16(var_v + _EPS)
            for j in range(_NS):
                sl = pl.ds(j * 16, 16)
                o_v[t, sl] = (o_v[t, sl] - mean_v) * inv * gamma_v[sl] + beta_v[sl]
            return _

        lax.fori_loop(0, _CHUNK, tok_body, None)
        pltpu.sync_copy(o_v, out_hbm.at[pl.ds(base, _CHUNK)])
        return _

    lax.fori_loop(0, n_chunks, chunk_body, None)


def _tc_part(x, ids, seg_table, gamma, beta):
    n_tok, h = x.shape
    nb = n_tok // _T
    ids3 = ids.reshape(nb, 1, _T)
    return pl.pallas_call(
        _tc_body,
        grid=(nb,),
        in_specs=[
            pl.BlockSpec((1, 1, _T), lambda i: (i, 0, 0)),
            pl.BlockSpec((_T, h), lambda i: (i, 0)),
            pl.BlockSpec((4, h), lambda i: (0, 0)),
            pl.BlockSpec((1, h), lambda i: (0, 0)),
            pl.BlockSpec((1, h), lambda i: (0, 0)),
        ],
        out_specs=pl.BlockSpec((_T, h), lambda i: (i, 0)),
        out_shape=jax.ShapeDtypeStruct((n_tok, h), jnp.float32),
    )(ids3, x, seg_table, gamma.reshape(1, h), beta.reshape(1, h))


def _sc_part(x, ids, seg_table, gamma, beta):
    n_tok, h = x.shape
    mesh = plsc.VectorSubcoreMesh(core_axis_name="c", subcore_axis_name="s")
    sck = functools.partial(
        pl.kernel,
        mesh=mesh,
        out_type=jax.ShapeDtypeStruct((n_tok, h), jnp.float32),
        scratch_types=[
            pltpu.VMEM((_CHUNK,), jnp.int32),
            pltpu.VMEM((_CHUNK, h), jnp.float32),
            pltpu.VMEM((_CHUNK, h), jnp.float32),
            pltpu.VMEM((_CHUNK, h), jnp.float32),
            pltpu.VMEM((h,), jnp.float32),
            pltpu.VMEM((h,), jnp.float32),
            pltpu.SemaphoreType.DMA,
        ],
    )(_sc_body)
    return sck(x, ids, seg_table, gamma, beta)


def kernel(input_embs, seg_ids, seg_table, ln_gamma, ln_beta):
    b, s, h = input_embs.shape
    n_tok = b * s
    x = input_embs.reshape(n_tok, h)
    ids = seg_ids.astype(jnp.int32).reshape(n_tok)
    n_tc = n_tok - _SC_TOK

    tc_out = _tc_part(x[:n_tc], ids[:n_tc], seg_table, ln_gamma, ln_beta)
    sc_out = _sc_part(x[n_tc:], ids[n_tc:], seg_table, ln_gamma, ln_beta)
    out = jnp.concatenate([tc_out, sc_out], axis=0)
    return out.reshape(b, s, h)


# speedup vs baseline: 11.6660x; 3.3379x over previous
import jax
import jax.numpy as jnp
from jax.experimental import pallas as pl

_HIDDEN = 768
_EPS = 1e-12
_T = 2048


def _fused_kernel(ids_ref, x_ref, table_ref, gamma_ref, beta_ref, out_ref):
    ids = ids_ref[0, 0, :]
    x = x_ref[...]
    table = table_ref[...]
    onehot = (ids[:, None] == jax.lax.broadcasted_iota(jnp.int32, (_T, 4), 1))
    seg = jnp.dot(onehot.astype(jnp.float32), table,
                  preferred_element_type=jnp.float32)
    e = x + seg
    mean = jnp.mean(e, axis=1, keepdims=True)
    d = e - mean
    var = jnp.mean(d * d, axis=1, keepdims=True)
    normed = d * jax.lax.rsqrt(var + _EPS)
    out_ref[...] = normed * gamma_ref[...] + beta_ref[...]


def kernel(input_embs, seg_ids, seg_table, ln_gamma, ln_beta):
    b, s, h = input_embs.shape
    n_tok = b * s
    nb = n_tok // _T
    x = input_embs.reshape(n_tok, h)
    ids = seg_ids.astype(jnp.int32).reshape(nb, 1, _T)
    gamma = ln_gamma.reshape(1, h)
    beta = ln_beta.reshape(1, h)

    out = pl.pallas_call(
        _fused_kernel,
        grid=(nb,),
        in_specs=[
            pl.BlockSpec((1, 1, _T), lambda i: (i, 0, 0)),
            pl.BlockSpec((_T, h), lambda i: (i, 0)),
            pl.BlockSpec((4, h), lambda i: (0, 0)),
            pl.BlockSpec((1, h), lambda i: (0, 0)),
            pl.BlockSpec((1, h), lambda i: (0, 0)),
        ],
        out_specs=pl.BlockSpec((_T, h), lambda i: (i, 0)),
        out_shape=jax.ShapeDtypeStruct((n_tok, h), jnp.float32),
    )(ids, x, seg_table, gamma, beta)
    return out.reshape(b, s, h)
